# Initial kernel scaffold; baseline (speedup 1.0000x reference)
import jax
import jax.numpy as jnp
from jax.experimental import pallas as pl


def kernel(x, dim, index, value, out):
    raise NotImplementedError("write your pallas kernel here")


# speedup vs baseline: 1.0136x; 1.0136x over previous
import functools

import jax
import jax.numpy as jnp
from jax import lax
from jax.experimental import pallas as pl
from jax.experimental.pallas import tpu as pltpu
from jax.experimental.pallas import tpu_sc as plsc

_N_ROWS = 100000
_D = 128
_NC = 2
_NS = 16
_NW = _NC * _NS
_RPW = 3200
_MASK_PAD = _NW * _RPW
_LANES = 16
_BLK = 5000


def _sc_build_mask(idx32):
    n_idx = idx32.shape[0]
    mesh = plsc.VectorSubcoreMesh(core_axis_name="c", subcore_axis_name="s")

    @functools.partial(
        pl.kernel,
        mesh=mesh,
        out_type=jax.ShapeDtypeStruct((_MASK_PAD,), jnp.float32),
        scratch_types=[
            pltpu.VMEM((n_idx,), jnp.int32),
            pltpu.VMEM((_RPW,), jnp.float32),
        ],
        compiler_params=pltpu.CompilerParams(needs_layout_passes=False),
    )
    def mask_kernel(idx_hbm, mask_hbm, idx_v, mask_v):
        wid = lax.axis_index("s") * _NC + lax.axis_index("c")
        base = wid * _RPW
        pltpu.sync_copy(idx_hbm, idx_v)

        zero = jnp.zeros((_LANES,), jnp.float32)

        def zero_body(i, c):
            mask_v[pl.ds(i * _LANES, _LANES)] = zero
            return c

        lax.fori_loop(0, _RPW // _LANES, zero_body, 0, unroll=8)

        one = jnp.ones((_LANES,), jnp.float32)

        def scan_body(j, c):
            rel = idx_v[pl.ds(j * _LANES, _LANES)] - base
            inb = (rel >= 0) & (rel < _RPW)
            plsc.store_scatter(mask_v, [rel], one, mask=inb)
            return c

        lax.fori_loop(0, n_idx // _LANES, scan_body, 0, unroll=8)

        pltpu.sync_copy(mask_v, mask_hbm.at[pl.ds(base, _RPW)])

    return mask_kernel(idx32)


def _tc_select(x, mask2d, value):
    def body(val_ref, mask_ref, x_ref, o_ref):
        o_ref[...] = jnp.where(mask_ref[...] != 0.0, val_ref[0], x_ref[...])

    return pl.pallas_call(
        body,
        grid=(_N_ROWS // _BLK,),
        in_specs=[
            pl.BlockSpec(memory_space=pltpu.SMEM),
            pl.BlockSpec((_BLK, 1), lambda i: (i, 0)),
            pl.BlockSpec((_BLK, _D), lambda i: (i, 0)),
        ],
        out_specs=pl.BlockSpec((_BLK, _D), lambda i: (i, 0)),
        out_shape=jax.ShapeDtypeStruct((_N_ROWS, _D), jnp.float32),
    )(jnp.reshape(value, (1,)), mask2d, x)


def kernel(x, dim, index, value, out):
    idx32 = (index + dim).astype(jnp.int32)
    mask = _sc_build_mask(idx32)
    return _tc_select(x, mask.reshape(_MASK_PAD, 1), value)
